# baseline (speedup 1.0000x reference)
import functools

import jax
import jax.numpy as jnp
from jax import lax
from jax.experimental import pallas as pl
from jax.experimental.pallas import tpu as pltpu
from jax.experimental.pallas import tpu_sc as plsc

K = 4
TC_ROWS = 32
N_CHUNKS = 16
N_WORKERS = 32
LANES = 16
LOAD_ROWS = 40


def _compute_chunk(buf, gbase, *, seq_len, hidden):
    n_col_groups = hidden // LANES
    is_batch_end = ((gbase + TC_ROWS) & (seq_len - 1)) == 0

    @pl.when(is_batch_end)
    def _zero_halo():
        def zcol(c, _):
            z = jnp.zeros((LANES,), jnp.float32)
            for r in range(K):
                buf[TC_ROWS + r, pl.ds(c * LANES, LANES)] = z
            return 0
        lax.fori_loop(0, n_col_groups, zcol, 0)

    def col_body(c, _):
        cs = c * LANES
        x1 = buf[1, pl.ds(cs, LANES)]
        x2 = buf[2, pl.ds(cs, LANES)]
        x3 = buf[3, pl.ds(cs, LANES)]
        q1 = x1 + x2
        q2 = x2 + x3
        xl = x3
        for t in range(TC_ROWS):
            v = buf[t + K, pl.ds(cs, LANES)]
            qn = xl + v
            buf[t, pl.ds(cs, LANES)] = (q1 + qn) * 0.25
            q1, q2, xl = q2, qn, v
        return 0

    lax.fori_loop(0, n_col_groups, col_body, 0)

    @pl.when(is_batch_end)
    def _fixup():
        factors = (4.0 / 3.0, 2.0, 4.0, 0.0)

        def fcol(c, _):
            cs = c * LANES
            for i, f in enumerate(factors):
                r = TC_ROWS - K + i
                buf[r, pl.ds(cs, LANES)] = buf[r, pl.ds(cs, LANES)] * f
            return 0
        lax.fori_loop(0, n_col_groups, fcol, 0)


def _start_load(h_hbm, buf, sem, gbase, n_rows):
    fits = gbase + LOAD_ROWS <= n_rows

    @pl.when(fits)
    def _one():
        pltpu.make_async_copy(
            h_hbm.at[pl.ds(gbase, LOAD_ROWS)],
            buf.at[pl.ds(0, LOAD_ROWS)], sem
        ).start()

    @pl.when(jnp.logical_not(fits))
    def _two():
        pltpu.make_async_copy(
            h_hbm.at[pl.ds(gbase, TC_ROWS)], buf.at[pl.ds(0, TC_ROWS)], sem
        ).start()
        pltpu.make_async_copy(
            h_hbm.at[pl.ds(n_rows - 8, K)], buf.at[pl.ds(TC_ROWS, K)], sem
        ).start()


def _wait_load(h_hbm, buf, sem, gbase, n_rows):
    fits = gbase + LOAD_ROWS <= n_rows

    @pl.when(fits)
    def _one():
        pltpu.make_async_copy(
            h_hbm.at[pl.ds(0, LOAD_ROWS)],
            buf.at[pl.ds(0, LOAD_ROWS)], sem
        ).wait()

    @pl.when(jnp.logical_not(fits))
    def _two():
        pltpu.make_async_copy(
            h_hbm.at[pl.ds(0, TC_ROWS)], buf.at[pl.ds(0, TC_ROWS)], sem
        ).wait()
        pltpu.make_async_copy(
            h_hbm.at[pl.ds(0, K)], buf.at[pl.ds(TC_ROWS, K)], sem
        ).wait()


def _start_scatter(out_hbm, buf, sem, gbase):
    pltpu.make_async_copy(
        buf.at[pl.ds(0, TC_ROWS)], out_hbm.at[pl.ds(gbase, TC_ROWS)], sem
    ).start()


def _wait_scatter(out_hbm, buf, sem):
    pltpu.make_async_copy(
        buf.at[pl.ds(0, TC_ROWS)], out_hbm.at[pl.ds(0, TC_ROWS)], sem
    ).wait()


def _sc_body(h_hbm, out_hbm, b0, b1, b2, l0, l1, l2, s0, s1, s2,
             *, n_rows, seq_len, hidden, rows_per_worker):
    nc = 2
    wid = lax.axis_index("s") * nc + lax.axis_index("c")
    base = wid * rows_per_worker
    bufs = (b0, b1, b2)
    lsems = (l0, l1, l2)
    ssems = (s0, s1, s2)
    compute = functools.partial(_compute_chunk, seq_len=seq_len, hidden=hidden)

    def run_chunk(c, slot):
        g = base + c * TC_ROWS
        pslot = (slot + 2) % 3

        _wait_load(h_hbm, bufs[slot], lsems[slot], g, n_rows)
        compute(bufs[slot], g)
        _start_scatter(out_hbm, bufs[slot], ssems[slot], g)

        @pl.when(c > 0)
        def _free_prev():
            _wait_scatter(out_hbm, bufs[pslot], ssems[pslot])

        @pl.when(c < N_CHUNKS - 2)
        def _prefetch():
            _start_load(h_hbm, bufs[pslot], lsems[pslot], g + 2 * TC_ROWS,
                        n_rows)

    _start_load(h_hbm, b0, l0, base, n_rows)
    _start_load(h_hbm, b1, l1, base + TC_ROWS, n_rows)

    def trip_body(t, _):
        c = 3 * t
        run_chunk(c, 0)
        run_chunk(c + 1, 1)
        run_chunk(c + 2, 2)
        return 0

    lax.fori_loop(0, (N_CHUNKS - 1) // 3, trip_body, 0)
    run_chunk(N_CHUNKS - 1, (N_CHUNKS - 1) % 3)
    _wait_scatter(out_hbm, bufs[(N_CHUNKS - 1) % 3], ssems[(N_CHUNKS - 1) % 3])


def kernel(hidden_states):
    B, S, H = hidden_states.shape
    n_rows = B * S
    rows_per_worker = n_rows // N_WORKERS
    flat = hidden_states.reshape(n_rows, H)

    mesh = plsc.VectorSubcoreMesh(core_axis_name="c", subcore_axis_name="s")
    body = functools.partial(
        _sc_body,
        n_rows=n_rows,
        seq_len=S,
        hidden=H,
        rows_per_worker=rows_per_worker,
    )
    run = pl.kernel(
        body,
        mesh=mesh,
        out_type=jax.ShapeDtypeStruct((n_rows, H), jnp.float32),
        scratch_types=[
            pltpu.VMEM((LOAD_ROWS, H), jnp.float32),
            pltpu.VMEM((LOAD_ROWS, H), jnp.float32),
            pltpu.VMEM((LOAD_ROWS, H), jnp.float32),
            pltpu.SemaphoreType.DMA,
            pltpu.SemaphoreType.DMA,
            pltpu.SemaphoreType.DMA,
            pltpu.SemaphoreType.DMA,
            pltpu.SemaphoreType.DMA,
            pltpu.SemaphoreType.DMA,
        ],
    )
    out = run(flat)
    return out.reshape(B, S, H)


# speedup vs baseline: 1.0367x; 1.0367x over previous
import functools

import jax
import jax.numpy as jnp
from jax import lax
from jax.experimental import pallas as pl
from jax.experimental.pallas import tpu as pltpu
from jax.experimental.pallas import tpu_sc as plsc

K = 4
TC_ROWS = 32
N_CHUNKS = 16
N_WORKERS = 32
LANES = 16


def _compute_chunk(buf, gbase, *, seq_len, hidden):
    n_col_groups = hidden // LANES
    is_batch_end = ((gbase + TC_ROWS) & (seq_len - 1)) == 0

    @pl.when(is_batch_end)
    def _zero_halo():
        def zcol(c, _):
            z = jnp.zeros((LANES,), jnp.float32)
            for r in range(K):
                buf[TC_ROWS + r, pl.ds(c * LANES, LANES)] = z
            return 0
        lax.fori_loop(0, n_col_groups, zcol, 0)

    def col_body(c, _):
        cs = c * LANES
        x1 = buf[1, pl.ds(cs, LANES)]
        x2 = buf[2, pl.ds(cs, LANES)]
        x3 = buf[3, pl.ds(cs, LANES)]
        q1 = x1 + x2
        q2 = x2 + x3
        xl = x3
        for t in range(TC_ROWS):
            v = buf[t + K, pl.ds(cs, LANES)]
            qn = xl + v
            buf[t, pl.ds(cs, LANES)] = (q1 + qn) * 0.25
            q1, q2, xl = q2, qn, v
        return 0

    lax.fori_loop(0, n_col_groups, col_body, 0)

    @pl.when(is_batch_end)
    def _fixup():
        factors = (4.0 / 3.0, 2.0, 4.0, 0.0)

        def fcol(c, _):
            cs = c * LANES
            for i, f in enumerate(factors):
                r = TC_ROWS - K + i
                buf[r, pl.ds(cs, LANES)] = buf[r, pl.ds(cs, LANES)] * f
            return 0
        lax.fori_loop(0, n_col_groups, fcol, 0)


def _start_load(h_hbm, buf, sem, gbase, n_rows):
    pltpu.make_async_copy(
        h_hbm.at[pl.ds(gbase, TC_ROWS)], buf.at[pl.ds(0, TC_ROWS)], sem
    ).start()
    hstart = jnp.minimum(gbase + TC_ROWS, n_rows - 8)
    pltpu.make_async_copy(
        h_hbm.at[pl.ds(hstart, K)], buf.at[pl.ds(TC_ROWS, K)], sem
    ).start()


def _wait_load(h_hbm, buf, sem, gbase, n_rows):
    pltpu.make_async_copy(
        h_hbm.at[pl.ds(0, TC_ROWS)], buf.at[pl.ds(0, TC_ROWS)], sem
    ).wait()
    pltpu.make_async_copy(
        h_hbm.at[pl.ds(0, K)], buf.at[pl.ds(TC_ROWS, K)], sem
    ).wait()


def _start_scatter(out_hbm, buf, sem, gbase):
    pltpu.make_async_copy(
        buf.at[pl.ds(0, TC_ROWS)], out_hbm.at[pl.ds(gbase, TC_ROWS)], sem
    ).start()


def _wait_scatter(out_hbm, buf, sem):
    pltpu.make_async_copy(
        buf.at[pl.ds(0, TC_ROWS)], out_hbm.at[pl.ds(0, TC_ROWS)], sem
    ).wait()


def _sc_body(h_hbm, out_hbm, b0, b1, b2, l0, l1, l2, s0, s1, s2,
             *, n_rows, seq_len, hidden, rows_per_worker):
    nc = 2
    wid = lax.axis_index("s") * nc + lax.axis_index("c")
    base = wid * rows_per_worker
    bufs = (b0, b1, b2)
    lsems = (l0, l1, l2)
    ssems = (s0, s1, s2)
    compute = functools.partial(_compute_chunk, seq_len=seq_len, hidden=hidden)

    def run_chunk(c, slot):
        g = base + c * TC_ROWS
        pslot = (slot + 2) % 3

        _wait_load(h_hbm, bufs[slot], lsems[slot], g, n_rows)
        compute(bufs[slot], g)
        _start_scatter(out_hbm, bufs[slot], ssems[slot], g)

        @pl.when(c > 0)
        def _free_prev():
            _wait_scatter(out_hbm, bufs[pslot], ssems[pslot])

        @pl.when(c < N_CHUNKS - 2)
        def _prefetch():
            _start_load(h_hbm, bufs[pslot], lsems[pslot], g + 2 * TC_ROWS,
                        n_rows)

    _start_load(h_hbm, b0, l0, base, n_rows)
    _start_load(h_hbm, b1, l1, base + TC_ROWS, n_rows)

    def trip_body(t, _):
        c = 3 * t
        run_chunk(c, 0)
        run_chunk(c + 1, 1)
        run_chunk(c + 2, 2)
        return 0

    lax.fori_loop(0, (N_CHUNKS - 1) // 3, trip_body, 0)
    run_chunk(N_CHUNKS - 1, (N_CHUNKS - 1) % 3)
    _wait_scatter(out_hbm, bufs[(N_CHUNKS - 1) % 3], ssems[(N_CHUNKS - 1) % 3])


def kernel(hidden_states):
    B, S, H = hidden_states.shape
    n_rows = B * S
    rows_per_worker = n_rows // N_WORKERS
    flat = hidden_states.reshape(n_rows, H)

    mesh = plsc.VectorSubcoreMesh(core_axis_name="c", subcore_axis_name="s")
    body = functools.partial(
        _sc_body,
        n_rows=n_rows,
        seq_len=S,
        hidden=H,
        rows_per_worker=rows_per_worker,
    )
    run = pl.kernel(
        body,
        mesh=mesh,
        out_type=jax.ShapeDtypeStruct((n_rows, H), jnp.float32),
        scratch_types=[
            pltpu.VMEM((TC_ROWS + K, H), jnp.float32),
            pltpu.VMEM((TC_ROWS + K, H), jnp.float32),
            pltpu.VMEM((TC_ROWS + K, H), jnp.float32),
            pltpu.SemaphoreType.DMA,
            pltpu.SemaphoreType.DMA,
            pltpu.SemaphoreType.DMA,
            pltpu.SemaphoreType.DMA,
            pltpu.SemaphoreType.DMA,
            pltpu.SemaphoreType.DMA,
        ],
    )
    out = run(flat)
    return out.reshape(B, S, H)


# speedup vs baseline: 1.0380x; 1.0012x over previous
import functools

import jax
import jax.numpy as jnp
from jax import lax
from jax.experimental import pallas as pl
from jax.experimental.pallas import tpu as pltpu
from jax.experimental.pallas import tpu_sc as plsc

K = 4
TC_ROWS = 32
N_CHUNKS = 16
N_WORKERS = 32
LANES = 16


def _compute_chunk(buf, gbase, *, seq_len, hidden):
    n_col_groups = hidden // LANES
    is_batch_end = ((gbase + TC_ROWS) & (seq_len - 1)) == 0

    @pl.when(is_batch_end)
    def _zero_halo():
        def zcol(c, _):
            z = jnp.zeros((LANES,), jnp.float32)
            for r in range(K):
                buf[TC_ROWS + r, pl.ds(c * LANES, LANES)] = z
            return 0
        lax.fori_loop(0, n_col_groups, zcol, 0)

    def make_col_body(t0, t1):
        def col_body(c, _):
            cs = c * LANES
            x1 = buf[t0 + 1, pl.ds(cs, LANES)]
            x2 = buf[t0 + 2, pl.ds(cs, LANES)]
            x3 = buf[t0 + 3, pl.ds(cs, LANES)]
            q1 = x1 + x2
            q2 = x2 + x3
            xl = x3
            for t in range(t0, t1):
                v = buf[t + K, pl.ds(cs, LANES)]
                qn = xl + v
                buf[t, pl.ds(cs, LANES)] = (q1 + qn) * 0.25
                q1, q2, xl = q2, qn, v
            return 0
        return col_body

    lax.fori_loop(0, n_col_groups, make_col_body(0, TC_ROWS // 2), 0)
    yield
    lax.fori_loop(0, n_col_groups, make_col_body(TC_ROWS // 2, TC_ROWS), 0)

    @pl.when(is_batch_end)
    def _fixup():
        factors = (4.0 / 3.0, 2.0, 4.0, 0.0)

        def fcol(c, _):
            cs = c * LANES
            for i, f in enumerate(factors):
                r = TC_ROWS - K + i
                buf[r, pl.ds(cs, LANES)] = buf[r, pl.ds(cs, LANES)] * f
            return 0
        lax.fori_loop(0, n_col_groups, fcol, 0)


def _start_load(h_hbm, buf, sem, gbase, n_rows):
    pltpu.make_async_copy(
        h_hbm.at[pl.ds(gbase, TC_ROWS)], buf.at[pl.ds(0, TC_ROWS)], sem
    ).start()
    hstart = jnp.minimum(gbase + TC_ROWS, n_rows - 8)
    pltpu.make_async_copy(
        h_hbm.at[pl.ds(hstart, K)], buf.at[pl.ds(TC_ROWS, K)], sem
    ).start()


def _wait_load(h_hbm, buf, sem, gbase, n_rows):
    pltpu.make_async_copy(
        h_hbm.at[pl.ds(0, TC_ROWS)], buf.at[pl.ds(0, TC_ROWS)], sem
    ).wait()
    pltpu.make_async_copy(
        h_hbm.at[pl.ds(0, K)], buf.at[pl.ds(TC_ROWS, K)], sem
    ).wait()


HALF = TC_ROWS // 2


def _start_scatter_half(out_hbm, buf, sem, gbase, half):
    r0 = half * HALF
    pltpu.make_async_copy(
        buf.at[pl.ds(r0, HALF)], out_hbm.at[pl.ds(gbase + r0, HALF)], sem
    ).start()


def _wait_scatter(out_hbm, buf, sem):
    for _ in range(2):
        pltpu.make_async_copy(
            buf.at[pl.ds(0, HALF)], out_hbm.at[pl.ds(0, HALF)], sem
        ).wait()


def _sc_body(h_hbm, out_hbm, b0, b1, b2, l0, l1, l2, s0, s1, s2,
             *, n_rows, seq_len, hidden, rows_per_worker):
    nc = 2
    wid = lax.axis_index("s") * nc + lax.axis_index("c")
    base = wid * rows_per_worker
    bufs = (b0, b1, b2)
    lsems = (l0, l1, l2)
    ssems = (s0, s1, s2)
    compute = functools.partial(_compute_chunk, seq_len=seq_len, hidden=hidden)

    def run_chunk(c, slot):
        g = base + c * TC_ROWS
        pslot = (slot + 2) % 3

        _wait_load(h_hbm, bufs[slot], lsems[slot], g, n_rows)
        gen = compute(bufs[slot], g)
        next(gen)
        _start_scatter_half(out_hbm, bufs[slot], ssems[slot], g, 0)
        for _ in gen:
            pass
        _start_scatter_half(out_hbm, bufs[slot], ssems[slot], g, 1)

        @pl.when(c > 0)
        def _free_prev():
            _wait_scatter(out_hbm, bufs[pslot], ssems[pslot])

        @pl.when(c < N_CHUNKS - 2)
        def _prefetch():
            _start_load(h_hbm, bufs[pslot], lsems[pslot], g + 2 * TC_ROWS,
                        n_rows)

    _start_load(h_hbm, b0, l0, base, n_rows)
    _start_load(h_hbm, b1, l1, base + TC_ROWS, n_rows)

    def trip_body(t, _):
        c = 3 * t
        run_chunk(c, 0)
        run_chunk(c + 1, 1)
        run_chunk(c + 2, 2)
        return 0

    lax.fori_loop(0, (N_CHUNKS - 1) // 3, trip_body, 0)
    run_chunk(N_CHUNKS - 1, (N_CHUNKS - 1) % 3)
    _wait_scatter(out_hbm, bufs[(N_CHUNKS - 1) % 3], ssems[(N_CHUNKS - 1) % 3])


def kernel(hidden_states):
    B, S, H = hidden_states.shape
    n_rows = B * S
    rows_per_worker = n_rows // N_WORKERS
    flat = hidden_states.reshape(n_rows, H)

    mesh = plsc.VectorSubcoreMesh(core_axis_name="c", subcore_axis_name="s")
    body = functools.partial(
        _sc_body,
        n_rows=n_rows,
        seq_len=S,
        hidden=H,
        rows_per_worker=rows_per_worker,
    )
    run = pl.kernel(
        body,
        mesh=mesh,
        out_type=jax.ShapeDtypeStruct((n_rows, H), jnp.float32),
        scratch_types=[
            pltpu.VMEM((TC_ROWS + K, H), jnp.float32),
            pltpu.VMEM((TC_ROWS + K, H), jnp.float32),
            pltpu.VMEM((TC_ROWS + K, H), jnp.float32),
            pltpu.SemaphoreType.DMA,
            pltpu.SemaphoreType.DMA,
            pltpu.SemaphoreType.DMA,
            pltpu.SemaphoreType.DMA,
            pltpu.SemaphoreType.DMA,
            pltpu.SemaphoreType.DMA,
        ],
    )
    out = run(flat)
    return out.reshape(B, S, H)
